# baseline (speedup 1.0000x reference)
import functools
import math

import jax
import jax.numpy as jnp
import numpy as np
from jax import lax
from jax.experimental import pallas as pl
from jax.experimental.pallas import tpu as pltpu
from jax.experimental.pallas import tpu_sc as plsc

_NBUF = 4


def _pe_table(seq_len: int, d_model: int) -> np.ndarray:
    pos = np.arange(seq_len)[:, None].astype(np.float32)
    div = np.exp(
        np.arange(0, d_model, 2).astype(np.float32) * -(math.log(10000.0) / d_model)
    )
    pe = np.zeros((seq_len, d_model), dtype=np.float32)
    pe[:, 0::2] = np.sin(pos * div)
    pe[:, 1::2] = np.cos(pos * div)
    return pe


@functools.cache
def _build_gather_scatter(batch: int, seq: int, vocab: int, d: int):
    n = batch * seq
    nc, ns, lanes = 2, 16, 16
    nw = nc * ns
    assert n % nw == 0 and d % lanes == 0
    per = n // nw
    chunk = seq
    nchunks = per // chunk
    assert nchunks % _NBUF == 0 and chunk <= 128
    nsteps = nchunks // _NBUF
    mesh = plsc.VectorSubcoreMesh(core_axis_name="c", subcore_axis_name="s")

    @functools.partial(
        pl.kernel,
        mesh=mesh,
        out_type=jax.ShapeDtypeStruct((n, d), jnp.float32),
        scratch_types=[
            pltpu.VMEM((nchunks, chunk), jnp.int32),
            pltpu.VMEM((nchunks, chunk), jnp.int32),
        ]
        + [pltpu.VMEM((chunk, d), jnp.float32)] * _NBUF
        + [pltpu.SemaphoreType.DMA] * (2 * _NBUF),
    )
    def gather_scatter(table, idx, oidx, out, idx_v, oidx_v, *bufs):
        gbufs = bufs[:_NBUF]
        gsems = bufs[_NBUF : 2 * _NBUF]
        osems = bufs[2 * _NBUF :]
        wid = lax.axis_index("s") * nc + lax.axis_index("c")
        pltpu.sync_copy(idx.at[wid], idx_v)
        pltpu.sync_copy(oidx.at[wid], oidx_v)

        for b in range(_NBUF):
            pltpu.async_copy(table.at[idx_v.at[b]], gbufs[b], gsems[b])

        def step(t, carry):
            j0 = t * _NBUF
            for b in range(_NBUF):
                pltpu.make_async_copy(
                    table.at[idx_v.at[0]], gbufs[b], gsems[b]
                ).wait()
                pltpu.async_copy(gbufs[b], out.at[oidx_v.at[j0 + b]], osems[b])

            @pl.when(t < nsteps - 1)
            def _():
                for b in range(_NBUF):
                    pltpu.make_async_copy(
                        gbufs[b], out.at[oidx_v.at[0]], osems[b]
                    ).wait()
                    pltpu.async_copy(
                        table.at[idx_v.at[j0 + _NBUF + b]], gbufs[b], gsems[b]
                    )
            return carry

        lax.fori_loop(0, nsteps, step, 0)
        for b in range(_NBUF):
            pltpu.make_async_copy(
                gbufs[b], out.at[oidx_v.at[0]], osems[b]
            ).wait()

    return gather_scatter, nw, nchunks, chunk


@functools.cache
def _build_epilogue(batch: int, seq: int, d: int, bb: int, half: int, first: bool):
    scale = np.float32(np.sqrt(np.float32(d)))
    off = 0 if first else (batch - half) // bb

    if first:

        def body(g_ref, pe_ref, o_ref):
            o_ref[...] = g_ref[...] * scale + pe_ref[...]

        extra_in = []
    else:

        def body(g_ref, pe_ref, prev_ref, o_ref):
            o_ref[...] = g_ref[...] * scale + pe_ref[...]

        extra_in = [pl.BlockSpec(memory_space=pltpu.MemorySpace.HBM)]

    return pl.pallas_call(
        body,
        grid=(half // bb,),
        in_specs=[
            pl.BlockSpec((seq, bb, d), lambda i: (0, i, 0)),
            pl.BlockSpec((seq, bb, d), lambda i: (0, 0, 0)),
        ]
        + extra_in,
        out_specs=pl.BlockSpec((seq, bb, d), lambda i: (0, i + off, 0)),
        out_shape=jax.ShapeDtypeStruct((seq, batch, d), jnp.float32),
        input_output_aliases={} if first else {2: 0},
    )


def kernel(text_ids, emb_table):
    batch, seq = text_ids.shape
    vocab, d = emb_table.shape
    half = batch // 2
    gather_scatter, nw, nchunks, chunk = _build_gather_scatter(half, seq, vocab, d)
    bb = 64
    epi1 = _build_epilogue(batch, seq, d, bb, half, True)
    epi2 = _build_epilogue(batch, seq, d, bb, half, False)
    pe_rep = jnp.asarray(np.tile(_pe_table(seq, d)[:, None, :], (1, bb, 1)))
    ids32 = text_ids.astype(jnp.int32)
    brow = np.arange(half, dtype=np.int32).reshape(nw, nchunks, 1)
    oidx = jnp.asarray(brow + half * np.arange(seq, dtype=np.int32)[None, None, :])
    inters = []
    for h in range(2):
        idx = ids32[h * half : (h + 1) * half].reshape(nw, nchunks, seq)
        inters.append(gather_scatter(emb_table, idx, oidx))
    o1 = epi1(inters[0].reshape(seq, half, d), pe_rep)
    o2 = epi2(inters[1].reshape(seq, half, d), pe_rep, o1)
    return o2.transpose(1, 0, 2)
